# baseline (speedup 1.0000x reference)
import functools
import math

import jax
import jax.numpy as jnp
from jax import lax
from jax.experimental import pallas as pl
from jax.experimental.pallas import tpu as pltpu
from jax.experimental.pallas import tpu_sc as plsc

D_MODEL = 128
LANES = 16
NUM_CORES = 2
NUM_SUBCORES = 16
NUM_WORKERS = NUM_CORES * NUM_SUBCORES
CHUNK = 64
SCALE = math.sqrt(float(D_MODEL))


@functools.lru_cache(maxsize=None)
def _build(b0: int, b1: int):
    batch = b0 * b1
    assert batch % (NUM_WORKERS * CHUNK) == 0
    bpw = batch // NUM_WORKERS
    nchunk = bpw // CHUNK
    assert b1 % bpw == 0
    tiles_per_row = b1 // bpw

    mesh = plsc.VectorSubcoreMesh(core_axis_name="c", subcore_axis_name="s")

    @functools.partial(
        pl.kernel,
        out_type=jax.ShapeDtypeStruct((batch, D_MODEL), jnp.float32),
        mesh=mesh,
        scratch_types=[
            pltpu.VMEM((bpw,), jnp.int32),
            pltpu.VMEM((bpw, D_MODEL), jnp.float32),
            pltpu.SemaphoreType.DMA,
            [pltpu.SemaphoreType.DMA] * 2,
            pltpu.SemaphoreType.DMA,
        ],
    )
    def emb_kernel(x_hbm, lut_hbm, out_hbm, idx_v, rows_v, isem, gsems, wsem):
        wid = lax.axis_index("s") * NUM_CORES + lax.axis_index("c")
        base = wid * bpw
        row = wid // tiles_per_row
        col = (wid % tiles_per_row) * bpw

        pltpu.async_copy(x_hbm.at[row, pl.ds(col, bpw)], idx_v, isem).wait()

        def gather(j, sem):
            off = j * CHUNK
            return pltpu.make_async_copy(
                lut_hbm.at[idx_v.at[pl.ds(off, CHUNK)]],
                rows_v.at[pl.ds(off, CHUNK)], sem)

        gather(0, gsems[0]).start()
        gather(1, gsems[1]).start()

        @pl.loop(0, nchunk // 2)
        def _(t):
            for p in range(2):
                j = 2 * t + p
                off = j * CHUNK
                gather(j, gsems[p]).wait()

                @plsc.parallel_loop(off, off + CHUNK, unroll=4)
                def _(r):
                    for c8 in range(D_MODEL // LANES):
                        sl = rows_v[r, pl.ds(c8 * LANES, LANES)]
                        rows_v[r, pl.ds(c8 * LANES, LANES)] = sl * SCALE

                pltpu.async_copy(rows_v.at[pl.ds(off, CHUNK)],
                                 out_hbm.at[pl.ds(base + off, CHUNK)], wsem)

                @pl.when(j + 2 < nchunk)
                def _():
                    gather(j + 2, gsems[p]).start()

        pltpu.make_async_copy(rows_v, out_hbm.at[pl.ds(base, bpw)],
                              wsem).wait()

    return emb_kernel


def kernel(x, lut):
    b0, b1 = x.shape
    if x.dtype != jnp.int32:
        x = x.astype(jnp.int32)
    out = _build(b0, b1)(x, lut)
    return out.reshape(b0, b1, D_MODEL)


# speedup vs baseline: 1.0010x; 1.0010x over previous
import functools
import math

import jax
import jax.numpy as jnp
from jax import lax
from jax.experimental import pallas as pl
from jax.experimental.pallas import tpu as pltpu
from jax.experimental.pallas import tpu_sc as plsc

D_MODEL = 128
LANES = 16
NUM_CORES = 2
NUM_SUBCORES = 16
NUM_WORKERS = NUM_CORES * NUM_SUBCORES
CHUNK = 64
NSEM = 4
SCALE = math.sqrt(float(D_MODEL))


@functools.lru_cache(maxsize=None)
def _build(b0: int, b1: int):
    batch = b0 * b1
    assert batch % (NUM_WORKERS * CHUNK) == 0
    bpw = batch // NUM_WORKERS
    nchunk = bpw // CHUNK
    assert b1 % bpw == 0
    tiles_per_row = b1 // bpw

    mesh = plsc.VectorSubcoreMesh(core_axis_name="c", subcore_axis_name="s")

    @functools.partial(
        pl.kernel,
        out_type=jax.ShapeDtypeStruct((batch, D_MODEL), jnp.float32),
        mesh=mesh,
        scratch_types=[
            pltpu.VMEM((bpw,), jnp.int32),
            pltpu.VMEM((bpw, D_MODEL), jnp.float32),
            pltpu.SemaphoreType.DMA,
            [pltpu.SemaphoreType.DMA] * NSEM,
            pltpu.SemaphoreType.DMA,
        ],
    )
    def emb_kernel(x_hbm, lut_hbm, out_hbm, idx_v, rows_v, isem, gsems, wsem):
        wid = lax.axis_index("s") * NUM_CORES + lax.axis_index("c")
        base = wid * bpw
        row = wid // tiles_per_row
        col = (wid % tiles_per_row) * bpw

        pltpu.async_copy(x_hbm.at[row, pl.ds(col, bpw)], idx_v, isem).wait()

        def gather(j):
            off = j * CHUNK
            return pltpu.make_async_copy(
                lut_hbm.at[idx_v.at[pl.ds(off, CHUNK)]],
                rows_v.at[pl.ds(off, CHUNK)], gsems[j % NSEM])

        for j in range(NSEM):
            gather(j).start()

        for j in range(nchunk):
            off = j * CHUNK
            gather(j).wait()

            @plsc.parallel_loop(off, off + CHUNK, unroll=4)
            def _(r):
                for c8 in range(D_MODEL // LANES):
                    sl = rows_v[r, pl.ds(c8 * LANES, LANES)]
                    rows_v[r, pl.ds(c8 * LANES, LANES)] = sl * SCALE

            pltpu.async_copy(rows_v.at[pl.ds(off, CHUNK)],
                             out_hbm.at[pl.ds(base + off, CHUNK)], wsem)
            if j + NSEM < nchunk:
                gather(j + NSEM).start()

        pltpu.make_async_copy(rows_v, out_hbm.at[pl.ds(base, bpw)],
                              wsem).wait()

    return emb_kernel


def kernel(x, lut):
    b0, b1 = x.shape
    if x.dtype != jnp.int32:
        x = x.astype(jnp.int32)
    out = _build(b0, b1)(x, lut)
    return out.reshape(b0, b1, D_MODEL)


# speedup vs baseline: 1.0644x; 1.0634x over previous
import functools
import math

import jax
import jax.numpy as jnp
from jax import lax
from jax.experimental import pallas as pl
from jax.experimental.pallas import tpu as pltpu
from jax.experimental.pallas import tpu_sc as plsc

D_MODEL = 128
LANES = 16
NUM_CORES = 2
NUM_SUBCORES = 16
NUM_WORKERS = NUM_CORES * NUM_SUBCORES
CHUNK = 64
NSEM = 8
SCALE = math.sqrt(float(D_MODEL))


@functools.lru_cache(maxsize=None)
def _build(b0: int, b1: int):
    batch = b0 * b1
    assert batch % (NUM_WORKERS * CHUNK) == 0
    bpw = batch // NUM_WORKERS
    nchunk = bpw // CHUNK
    assert b1 % bpw == 0
    tiles_per_row = b1 // bpw

    mesh = plsc.VectorSubcoreMesh(core_axis_name="c", subcore_axis_name="s")

    @functools.partial(
        pl.kernel,
        out_type=jax.ShapeDtypeStruct((batch, D_MODEL), jnp.float32),
        mesh=mesh,
        scratch_types=[
            pltpu.VMEM((bpw,), jnp.int32),
            pltpu.VMEM((bpw, D_MODEL), jnp.float32),
            pltpu.SemaphoreType.DMA,
            [pltpu.SemaphoreType.DMA] * NSEM,
            pltpu.SemaphoreType.DMA,
        ],
    )
    def emb_kernel(x_hbm, lut_hbm, out_hbm, idx_v, rows_v, isem, gsems, wsem):
        wid = lax.axis_index("s") * NUM_CORES + lax.axis_index("c")
        base = wid * bpw
        row = wid // tiles_per_row
        col = (wid % tiles_per_row) * bpw

        pltpu.async_copy(x_hbm.at[row, pl.ds(col, bpw)], idx_v, isem).wait()

        def gather(j):
            off = j * CHUNK
            return pltpu.make_async_copy(
                lut_hbm.at[idx_v.at[pl.ds(off, CHUNK)]],
                rows_v.at[pl.ds(off, CHUNK)], gsems[j % NSEM])

        for j in range(NSEM):
            gather(j).start()

        for j in range(nchunk):
            off = j * CHUNK
            gather(j).wait()

            @plsc.parallel_loop(off, off + CHUNK, unroll=1)
            def _(r):
                for c8 in range(D_MODEL // LANES):
                    sl = rows_v[r, pl.ds(c8 * LANES, LANES)]
                    rows_v[r, pl.ds(c8 * LANES, LANES)] = sl * SCALE

            pltpu.async_copy(rows_v.at[pl.ds(off, CHUNK)],
                             out_hbm.at[pl.ds(base + off, CHUNK)], wsem)
            if j + NSEM < nchunk:
                gather(j + NSEM).start()

        pltpu.make_async_copy(rows_v, out_hbm.at[pl.ds(base, bpw)],
                              wsem).wait()

    return emb_kernel


def kernel(x, lut):
    b0, b1 = x.shape
    if x.dtype != jnp.int32:
        x = x.astype(jnp.int32)
    out = _build(b0, b1)(x, lut)
    return out.reshape(b0, b1, D_MODEL)
